# baseline (speedup 1.0000x reference)
import functools

import jax
import jax.numpy as jnp
from jax import lax
from jax.experimental import pallas as pl
from jax.experimental.pallas import tpu as pltpu
from jax.experimental.pallas import tpu_sc as plsc

N = 10000
E = 320000
D = 128
U = 1000
NC = 2
NS = 16
NW = NC * NS
EPW = E // NW
K = 100
NCH = EPW // K
NP = 10240
EPP = 10240
SREG = 10256
N2 = 1024

_mesh = plsc.VectorSubcoreMesh(core_axis_name="c", subcore_axis_name="s")


def _pos_body(d_ref, pos_ref, cnt_ref):
    w = pl.program_id(0)
    sidw = lax.rem(w, NS)
    m = (d_ref[0] < U).astype(jnp.float32)
    r1 = lax.broadcasted_iota(jnp.int32, (128, 128), 0)
    c1 = lax.broadcasted_iota(jnp.int32, (128, 128), 1)
    ut = (r1 <= c1).astype(jnp.float32)
    incl = jnp.dot(m, ut, preferred_element_type=jnp.float32)
    rowtot = incl[:, 127:128]
    r2 = lax.broadcasted_iota(jnp.int32, (80, 80), 0)
    c2 = lax.broadcasted_iota(jnp.int32, (80, 80), 1)
    lst = (c2 < r2).astype(jnp.float32)
    rowbase = jnp.dot(lst, rowtot, preferred_element_type=jnp.float32)
    excl = rowbase + incl - m
    sbase = (sidw * SREG).astype(jnp.float32)
    dump = sbase + float(EPP)
    posf = jnp.where(m > 0, excl + sbase, dump)
    pos_ref[0] = posf.astype(jnp.int32)
    total = jnp.sum(m)
    cnt_ref[...] = jnp.full((1, 8, 128), total).astype(jnp.int32)


@functools.partial(
    pl.kernel,
    out_type=(
        jax.ShapeDtypeStruct((NC, N), jnp.float32),
        jax.ShapeDtypeStruct((NC, NS * SREG), jnp.int32),
        jax.ShapeDtypeStruct((NC, NS * SREG), jnp.int32),
    ),
    mesh=_mesh,
    scratch_types=[
        pltpu.VMEM((NCH, K), jnp.int32),
        pltpu.VMEM((EPP // 128, 128), jnp.int32),
        pltpu.VMEM((EPP // 128, 128), jnp.int32),
        pltpu.VMEM((EPP // 128, 128), jnp.int32),
        pltpu.VMEM((8, 128), jnp.int32),
        pltpu.VMEM((1, 128), jnp.int32),
        pltpu.VMEM((128,), jnp.int32),
        pltpu.VMEM((128,), jnp.int32),
        pltpu.VMEM((112,), jnp.float32),
        pltpu.VMEM((2000,), jnp.float32),
        pltpu.VMEM_SHARED((N,), jnp.float32),
        pltpu.VMEM_SHARED((NS * SREG,), jnp.int32),
        pltpu.VMEM_SHARED((NS * SREG,), jnp.int32),
    ],
)
def _prep_kernel(dst1_hbm, srcv_hbm, dstv_hbm, pos_hbm, cnt_hbm,
                 deg_hbm, csrc_hbm, cdst_hbm,
                 idx_v, sv_v, dv_v, pos_v, cnt_v, iw_v, ssent_v, dsent_v,
                 ones_v, zb_v, acc_sh, cbs_sh, cbd_sh):
    cid = lax.axis_index("c")
    sid = lax.axis_index("s")
    pltpu.sync_copy(dst1_hbm.at[cid, sid], idx_v)
    pltpu.sync_copy(srcv_hbm.at[cid, sid], sv_v)
    pltpu.sync_copy(dstv_hbm.at[cid, sid], dv_v)
    pltpu.sync_copy(pos_hbm.at[cid, sid], pos_v)
    pltpu.sync_copy(cnt_hbm.at[cid, sid], cnt_v)

    for i in range(112 // 16):
        ones_v[pl.ds(i * 16, 16)] = jnp.ones((16,), jnp.float32)
    for i in range(128 // 16):
        ssent_v[pl.ds(i * 16, 16)] = jnp.full((16,), N, jnp.int32)
        dsent_v[pl.ds(i * 16, 16)] = jnp.full((16,), U, jnp.int32)

    @pl.when(sid == 0)
    def _():
        z = jnp.zeros((16,), jnp.float32)

        @pl.loop(0, 125)
        def _(r):
            zb_v[pl.ds(r * 16, 16)] = z

        for i in range(5):
            pltpu.sync_copy(zb_v, acc_sh.at[pl.ds(i * 2000, 2000)])

    @pl.loop(0, EPP // 128)
    def _(j):
        pltpu.sync_copy(sv_v.at[j], cbs_sh.at[pos_v.at[j]])
        pltpu.sync_copy(dv_v.at[j], cbd_sh.at[pos_v.at[j]])

    cnt = cnt_v[0, pl.ds(0, 16)][0]
    base = sid * SREG + cnt
    for kk in range(128 // 16):
        iw_v[0, pl.ds(kk * 16, 16)] = (
            lax.iota(jnp.int32, 16) + (base + kk * 16))
    pltpu.sync_copy(ssent_v, cbs_sh.at[iw_v.at[0]])
    pltpu.sync_copy(dsent_v, cbd_sh.at[iw_v.at[0]])

    plsc.subcore_barrier()

    @pl.when(sid == 1)
    def _():
        pltpu.sync_copy(cbs_sh, csrc_hbm.at[cid])

    @pl.when(sid == 2)
    def _():
        pltpu.sync_copy(cbd_sh, cdst_hbm.at[cid])

    @pl.loop(0, NCH)
    def _(j):
        pltpu.sync_copy(ones_v.at[pl.ds(0, K)], acc_sh.at[idx_v.at[j]],
                        add=True)

    plsc.subcore_barrier()

    @pl.when(sid == 0)
    def _():
        pltpu.sync_copy(acc_sh, deg_hbm.at[cid])


@functools.partial(
    pl.kernel,
    out_type=jax.ShapeDtypeStruct((NC, NP, D), jnp.float32),
    mesh=_mesh,
    scratch_types=[
        pltpu.VMEM((52, K), jnp.int32),
        pltpu.VMEM((52, K), jnp.int32),
        pltpu.VMEM((2, K, D), jnp.float32),
        pltpu.VMEM_SHARED((NP, D), jnp.float32),
        pltpu.SemaphoreType.DMA,
        pltpu.SemaphoreType.DMA,
    ],
)
def _edge_kernel(y_hbm, src_hbm, dst_hbm, out_hbm, sidx_v, didx_v, buf_v,
                 acc_sh, sem0, sem1):
    cid = lax.axis_index("c")
    sid = lax.axis_index("s")
    sems = (sem0, sem1)

    z = jnp.zeros((16,), jnp.float32)

    @pl.loop(0, 16)
    def _(r):
        for c in range(D // 16):
            buf_v[0, r, pl.ds(c * 16, 16)] = z

    @pl.loop(sid, NP // 16, step=NS)
    def _(i):
        pltpu.sync_copy(buf_v.at[0, pl.ds(0, 16)], acc_sh.at[pl.ds(i * 16, 16)])

    plsc.subcore_barrier()

    def issue(j, b):
        pltpu.async_copy(y_hbm.at[sidx_v.at[j]], buf_v.at[b], sems[b])

    def wait_data(b):
        pltpu.make_async_copy(y_hbm.at[sidx_v.at[0]], buf_v.at[b],
                              sems[b]).wait()

    for off, hc in ((0, 48), (48, 52)):
        pltpu.sync_copy(src_hbm.at[cid, sid, pl.ds(off, hc)],
                        sidx_v.at[pl.ds(0, hc)])
        pltpu.sync_copy(dst_hbm.at[cid, sid, pl.ds(off, hc)],
                        didx_v.at[pl.ds(0, hc)])
        issue(0, 0)
        issue(1, 1)

        @pl.loop(0, hc - 2, step=2)
        def _(j):
            for b in range(2):
                wait_data(b)
                pltpu.sync_copy(buf_v.at[b], acc_sh.at[didx_v.at[j + b]],
                                add=True)
                issue(j + b + 2, b)

        for b in range(2):
            wait_data(b)
            pltpu.sync_copy(buf_v.at[b], acc_sh.at[didx_v.at[hc - 2 + b]],
                            add=True)

    plsc.subcore_barrier()

    @pl.loop(sid, NP // 16, step=NS)
    def _(i):
        pltpu.sync_copy(acc_sh.at[pl.ds(i * 16, 16)],
                        out_hbm.at[cid, pl.ds(i * 16, 16)])


@functools.partial(
    pl.kernel,
    out_type=jax.ShapeDtypeStruct((NC, N2, D), jnp.float32),
    mesh=_mesh,
    scratch_types=[
        pltpu.VMEM((NCH, K), jnp.int32),
        pltpu.VMEM((NCH, K), jnp.int32),
        pltpu.VMEM((8, 128), jnp.int32),
        pltpu.VMEM((2, K, D), jnp.float32),
        pltpu.VMEM_SHARED((N2, D), jnp.float32),
        pltpu.SemaphoreType.DMA,
        pltpu.SemaphoreType.DMA,
    ],
)
def _edge2_kernel(y_hbm, src_hbm, dst_hbm, cnt_hbm, out_hbm, sidx_v, didx_v,
                  cnt_v, buf_v, acc_sh, sem0, sem1):
    cid = lax.axis_index("c")
    sid = lax.axis_index("s")
    sems = (sem0, sem1)
    pltpu.sync_copy(src_hbm.at[cid, sid], sidx_v)
    pltpu.sync_copy(dst_hbm.at[cid, sid], didx_v)
    pltpu.sync_copy(cnt_hbm.at[cid, sid], cnt_v)
    cnt = cnt_v[0, pl.ds(0, 16)][0]
    ncw = (cnt + (K - 1)) // K

    z = jnp.zeros((16,), jnp.float32)

    @pl.loop(0, 16)
    def _(r):
        for c in range(D // 16):
            buf_v[0, r, pl.ds(c * 16, 16)] = z

    @pl.loop(sid, N2 // 16, step=NS)
    def _(i):
        pltpu.sync_copy(buf_v.at[0, pl.ds(0, 16)], acc_sh.at[pl.ds(i * 16, 16)])

    plsc.subcore_barrier()

    def issue(j, b):
        pltpu.async_copy(y_hbm.at[sidx_v.at[j]], buf_v.at[b], sems[b])

    def wait_data(b):
        pltpu.make_async_copy(y_hbm.at[sidx_v.at[0]], buf_v.at[b],
                              sems[b]).wait()

    @pl.when(0 < ncw)
    def _():
        issue(0, 0)

    @pl.when(1 < ncw)
    def _():
        issue(1, 1)

    @pl.loop(0, (ncw + 1) // 2)
    def _(t):
        j0 = 2 * t
        for b in range(2):
            j = j0 + b

            @pl.when(j < ncw)
            def _():
                wait_data(b)
                pltpu.sync_copy(buf_v.at[b], acc_sh.at[didx_v.at[j]],
                                add=True)

                @pl.when(j + 2 < ncw)
                def _():
                    issue(j + 2, b)

    plsc.subcore_barrier()

    @pl.loop(sid, N2 // 16, step=NS)
    def _(i):
        pltpu.sync_copy(acc_sh.at[pl.ds(i * 16, 16)],
                        out_hbm.at[cid, pl.ds(i * 16, 16)])


def _y1_body(x_ref, w_ref, d0_ref, d1_ref, y_ref):
    dis = lax.rsqrt(d0_ref[...] + d1_ref[...] + 1.0)
    xw = jnp.dot(x_ref[...], w_ref[...], preferred_element_type=jnp.float32)
    y_ref[...] = xw * dis


def _mid_body(p0_ref, p1_ref, y_ref, d0_ref, d1_ref, b_ref, w_ref, out_ref):
    dis = lax.rsqrt(d0_ref[...] + d1_ref[...] + 1.0)
    t = dis * (p0_ref[...] + p1_ref[...] + y_ref[...]) + b_ref[...]
    h = jnp.where(t > 0, t, jnp.exp(t) - 1.0)
    hw = jnp.dot(h, w_ref[...], preferred_element_type=jnp.float32)
    out_ref[...] = hw * dis


def _final_body(q0_ref, q1_ref, y_ref, d0_ref, d1_ref, b_ref, out_ref):
    dis = lax.rsqrt(d0_ref[...] + d1_ref[...] + 1.0)
    t = dis * (q0_ref[...] + q1_ref[...] + y_ref[...]) + b_ref[...]
    out_ref[...] = jnp.where(t > 0, t, jnp.exp(t) - 1.0)


def kernel(x, edge_index, W1, b1, W2, b2):
    ei = edge_index.astype(jnp.int32)
    src = ei[0]
    dst = ei[1]
    src1 = src.reshape(NC, NS, NCH, K)
    dst1 = dst.reshape(NC, NS, NCH, K)

    srcp = jnp.concatenate(
        [src.reshape(NW, EPW), jnp.full((NW, EPP - EPW), N, jnp.int32)],
        axis=1)
    dstp = jnp.concatenate(
        [dst.reshape(NW, EPW), jnp.full((NW, EPP - EPW), U, jnp.int32)],
        axis=1)
    src4 = srcp.reshape(NW, EPP // 128, 128)
    dst4 = dstp.reshape(NW, EPP // 128, 128)

    pos4, counts = pl.pallas_call(
        _pos_body,
        grid=(NW,),
        in_specs=[pl.BlockSpec((1, EPP // 128, 128), lambda i: (i, 0, 0))],
        out_specs=[pl.BlockSpec((1, EPP // 128, 128), lambda i: (i, 0, 0)),
                   pl.BlockSpec((1, 8, 128), lambda i: (i, 0, 0))],
        out_shape=[jax.ShapeDtypeStruct((NW, EPP // 128, 128), jnp.int32),
                   jax.ShapeDtypeStruct((NW, 8, 128), jnp.int32)],
    )(dst4)

    cnts = counts.reshape(NC, NS, 8, 128)
    deg_p, csrc, cdst = _prep_kernel(
        dst1, src4.reshape(NC, NS, EPP // 128, 128),
        dst4.reshape(NC, NS, EPP // 128, 128),
        pos4.reshape(NC, NS, EPP // 128, 128), cnts)

    degpad = jnp.zeros((NC, NP - N), jnp.float32)
    degp = jnp.concatenate([deg_p, degpad], axis=1)
    d0 = degp[0].reshape(NP, 1)
    d1 = degp[1].reshape(NP, 1)

    xp = jnp.concatenate([x, jnp.zeros((NP - N, D), x.dtype)], axis=0)

    R = 320
    grid = NP // R
    row_spec = pl.BlockSpec((R, D), lambda i: (i, 0))
    dcol_spec = pl.BlockSpec((R, 1), lambda i: (i, 0))
    full_spec = pl.BlockSpec((D, D), lambda i: (0, 0))
    bias_spec = pl.BlockSpec((1, D), lambda i: (0, 0))

    y1 = pl.pallas_call(
        _y1_body,
        grid=(grid,),
        in_specs=[row_spec, full_spec, dcol_spec, dcol_spec],
        out_specs=row_spec,
        out_shape=jax.ShapeDtypeStruct((NP, D), jnp.float32),
    )(xp, W1, d0, d1)

    p = _edge_kernel(y1, src1, dst1)

    y2 = pl.pallas_call(
        _mid_body,
        grid=(grid,),
        in_specs=[row_spec, row_spec, row_spec, dcol_spec, dcol_spec,
                  bias_spec, full_spec],
        out_specs=row_spec,
        out_shape=jax.ShapeDtypeStruct((NP, D), jnp.float32),
    )(p[0], p[1], y1, d0, d1, b1.reshape(1, D), W2)

    csrc2 = csrc.reshape(NC, NS, SREG)[:, :, :EPW].reshape(NC, NS, NCH, K)
    cdst2 = cdst.reshape(NC, NS, SREG)[:, :, :EPW].reshape(NC, NS, NCH, K)
    q = _edge2_kernel(y2, csrc2, cdst2, cnts)

    RT = 128
    top_spec = pl.BlockSpec((RT, D), lambda i: (i, 0))
    top_dcol = pl.BlockSpec((RT, 1), lambda i: (i, 0))
    top_bias = pl.BlockSpec((1, D), lambda i: (0, 0))
    out_top = pl.pallas_call(
        _final_body,
        grid=(N2 // RT,),
        in_specs=[top_spec, top_spec, top_spec, top_dcol, top_dcol, top_bias],
        out_specs=top_spec,
        out_shape=jax.ShapeDtypeStruct((N2, D), jnp.float32),
    )(q[0], q[1], y2[:N2], d0[:N2], d1[:N2], b2.reshape(1, D))

    return jnp.concatenate([out_top[:U], x[U:]], axis=0)


# speedup vs baseline: 1.0039x; 1.0039x over previous
import functools

import jax
import jax.numpy as jnp
from jax import lax
from jax.experimental import pallas as pl
from jax.experimental.pallas import tpu as pltpu
from jax.experimental.pallas import tpu_sc as plsc

N = 10000
E = 320000
D = 128
U = 1000
NC = 2
NS = 16
NW = NC * NS
EPW = E // NW
K = 100
NCH = EPW // K
NP = 10240
EPP = 10240
SREG = 10256
N2 = 1024

_mesh = plsc.VectorSubcoreMesh(core_axis_name="c", subcore_axis_name="s")


def _pos_body(d_ref, pos_ref, cnt_ref):
    w = pl.program_id(0)
    sidw = lax.rem(w, NS)
    m = (d_ref[0] < U).astype(jnp.float32)
    r1 = lax.broadcasted_iota(jnp.int32, (128, 128), 0)
    c1 = lax.broadcasted_iota(jnp.int32, (128, 128), 1)
    ut = (r1 <= c1).astype(jnp.float32)
    incl = jnp.dot(m, ut, preferred_element_type=jnp.float32)
    rowtot = incl[:, 127:128]
    r2 = lax.broadcasted_iota(jnp.int32, (80, 80), 0)
    c2 = lax.broadcasted_iota(jnp.int32, (80, 80), 1)
    lst = (c2 < r2).astype(jnp.float32)
    rowbase = jnp.dot(lst, rowtot, preferred_element_type=jnp.float32)
    excl = rowbase + incl - m
    sbase = (sidw * SREG).astype(jnp.float32)
    dump = sbase + float(EPP)
    posf = jnp.where(m > 0, excl + sbase, dump)
    pos_ref[0] = posf.astype(jnp.int32)
    total = jnp.sum(m)
    cnt_ref[...] = jnp.full((1, 8, 128), total).astype(jnp.int32)


@functools.partial(
    pl.kernel,
    out_type=(
        jax.ShapeDtypeStruct((NC, N), jnp.float32),
        jax.ShapeDtypeStruct((NC, NS * SREG), jnp.int32),
        jax.ShapeDtypeStruct((NC, NS * SREG), jnp.int32),
    ),
    mesh=_mesh,
    scratch_types=[
        pltpu.VMEM((NCH, K), jnp.int32),
        pltpu.VMEM((EPP // 128, 128), jnp.int32),
        pltpu.VMEM((EPP // 128, 128), jnp.int32),
        pltpu.VMEM((EPP // 128, 128), jnp.int32),
        pltpu.VMEM((8, 128), jnp.int32),
        pltpu.VMEM((1, 128), jnp.int32),
        pltpu.VMEM((128,), jnp.int32),
        pltpu.VMEM((128,), jnp.int32),
        pltpu.VMEM((112,), jnp.float32),
        pltpu.VMEM((2000,), jnp.float32),
        pltpu.VMEM_SHARED((N,), jnp.float32),
        pltpu.VMEM_SHARED((NS * SREG,), jnp.int32),
        pltpu.VMEM_SHARED((NS * SREG,), jnp.int32),
    ],
)
def _prep_kernel(dst1_hbm, srcv_hbm, dstv_hbm, pos_hbm, cnt_hbm,
                 deg_hbm, csrc_hbm, cdst_hbm,
                 idx_v, sv_v, dv_v, pos_v, cnt_v, iw_v, ssent_v, dsent_v,
                 ones_v, zb_v, acc_sh, cbs_sh, cbd_sh):
    cid = lax.axis_index("c")
    sid = lax.axis_index("s")
    pltpu.sync_copy(dst1_hbm.at[cid, sid], idx_v)
    pltpu.sync_copy(srcv_hbm.at[cid, sid], sv_v)
    pltpu.sync_copy(dstv_hbm.at[cid, sid], dv_v)
    pltpu.sync_copy(pos_hbm.at[cid, sid], pos_v)
    pltpu.sync_copy(cnt_hbm.at[cid, sid], cnt_v)

    for i in range(112 // 16):
        ones_v[pl.ds(i * 16, 16)] = jnp.ones((16,), jnp.float32)
    for i in range(128 // 16):
        ssent_v[pl.ds(i * 16, 16)] = jnp.full((16,), N, jnp.int32)
        dsent_v[pl.ds(i * 16, 16)] = jnp.full((16,), U, jnp.int32)

    @pl.when(sid == 0)
    def _():
        z = jnp.zeros((16,), jnp.float32)

        @pl.loop(0, 125)
        def _(r):
            zb_v[pl.ds(r * 16, 16)] = z

        for i in range(5):
            pltpu.sync_copy(zb_v, acc_sh.at[pl.ds(i * 2000, 2000)])

    @pl.loop(0, EPP // 128)
    def _(j):
        pltpu.sync_copy(sv_v.at[j], cbs_sh.at[pos_v.at[j]])
        pltpu.sync_copy(dv_v.at[j], cbd_sh.at[pos_v.at[j]])

    cnt = cnt_v[0, pl.ds(0, 16)][0]
    base = sid * SREG + cnt
    for kk in range(128 // 16):
        iw_v[0, pl.ds(kk * 16, 16)] = (
            lax.iota(jnp.int32, 16) + (base + kk * 16))
    pltpu.sync_copy(ssent_v, cbs_sh.at[iw_v.at[0]])
    pltpu.sync_copy(dsent_v, cbd_sh.at[iw_v.at[0]])

    plsc.subcore_barrier()

    @pl.when(sid == 1)
    def _():
        pltpu.sync_copy(cbs_sh, csrc_hbm.at[cid])

    @pl.when(sid == 2)
    def _():
        pltpu.sync_copy(cbd_sh, cdst_hbm.at[cid])

    @pl.loop(0, NCH)
    def _(j):
        pltpu.sync_copy(ones_v.at[pl.ds(0, K)], acc_sh.at[idx_v.at[j]],
                        add=True)

    plsc.subcore_barrier()

    @pl.when(sid == 0)
    def _():
        pltpu.sync_copy(acc_sh, deg_hbm.at[cid])


@functools.partial(
    pl.kernel,
    out_type=jax.ShapeDtypeStruct((NC, NP, D), jnp.float32),
    mesh=_mesh,
    scratch_types=[
        pltpu.VMEM((52, K), jnp.int32),
        pltpu.VMEM((52, K), jnp.int32),
        pltpu.VMEM((2, K, D), jnp.float32),
        pltpu.VMEM_SHARED((NP, D), jnp.float32),
        pltpu.SemaphoreType.DMA,
        pltpu.SemaphoreType.DMA,
    ],
)
def _edge_kernel(y_hbm, src_hbm, dst_hbm, out_hbm, sidx_v, didx_v, buf_v,
                 acc_sh, sem0, sem1):
    cid = lax.axis_index("c")
    sid = lax.axis_index("s")
    sems = (sem0, sem1)

    z = jnp.zeros((16,), jnp.float32)

    @pl.loop(0, 16)
    def _(r):
        for c in range(D // 16):
            buf_v[0, r, pl.ds(c * 16, 16)] = z

    @pl.loop(sid, NP // 16, step=NS)
    def _(i):
        pltpu.sync_copy(buf_v.at[0, pl.ds(0, 16)], acc_sh.at[pl.ds(i * 16, 16)])

    plsc.subcore_barrier()

    def issue(j, b):
        pltpu.async_copy(y_hbm.at[sidx_v.at[j]], buf_v.at[b], sems[b])

    def wait_data(b):
        pltpu.make_async_copy(y_hbm.at[sidx_v.at[0]], buf_v.at[b],
                              sems[b]).wait()

    for off, hc in ((0, 48), (48, 52)):
        pltpu.sync_copy(src_hbm.at[cid, sid, pl.ds(off, hc)],
                        sidx_v.at[pl.ds(0, hc)])
        pltpu.sync_copy(dst_hbm.at[cid, sid, pl.ds(off, hc)],
                        didx_v.at[pl.ds(0, hc)])
        issue(0, 0)
        issue(1, 1)

        @pl.loop(0, hc - 2, step=2)
        def _(j):
            for b in range(2):
                wait_data(b)
                pltpu.sync_copy(buf_v.at[b], acc_sh.at[didx_v.at[j + b]],
                                add=True)
                issue(j + b + 2, b)

        for b in range(2):
            wait_data(b)
            pltpu.sync_copy(buf_v.at[b], acc_sh.at[didx_v.at[hc - 2 + b]],
                            add=True)

    plsc.subcore_barrier()

    @pl.loop(sid, NP // 16, step=NS)
    def _(i):
        pltpu.sync_copy(acc_sh.at[pl.ds(i * 16, 16)],
                        out_hbm.at[cid, pl.ds(i * 16, 16)])


@functools.partial(
    pl.kernel,
    out_type=jax.ShapeDtypeStruct((NC, N2, D), jnp.float32),
    mesh=_mesh,
    scratch_types=[
        pltpu.VMEM((NCH, K), jnp.int32),
        pltpu.VMEM((NCH, K), jnp.int32),
        pltpu.VMEM((8, 128), jnp.int32),
        pltpu.VMEM((2, K, D), jnp.float32),
        pltpu.VMEM_SHARED((N2, D), jnp.float32),
        pltpu.SemaphoreType.DMA,
        pltpu.SemaphoreType.DMA,
    ],
)
def _edge2_kernel(y_hbm, src_hbm, dst_hbm, cnt_hbm, out_hbm, sidx_v, didx_v,
                  cnt_v, buf_v, acc_sh, sem0, sem1):
    cid = lax.axis_index("c")
    sid = lax.axis_index("s")
    sems = (sem0, sem1)
    pltpu.sync_copy(cnt_hbm.at[cid, sid], cnt_v)
    cnt = cnt_v[0, pl.ds(0, 16)][0]
    ncw = (cnt + (K - 1)) // K
    pltpu.sync_copy(src_hbm.at[cid, sid, pl.ds(0, 16)],
                    sidx_v.at[pl.ds(0, 16)])
    pltpu.sync_copy(dst_hbm.at[cid, sid, pl.ds(0, 16)],
                    didx_v.at[pl.ds(0, 16)])

    @pl.when(ncw > 16)
    def _():
        pltpu.sync_copy(src_hbm.at[cid, sid, pl.ds(16, NCH - 16)],
                        sidx_v.at[pl.ds(16, NCH - 16)])
        pltpu.sync_copy(dst_hbm.at[cid, sid, pl.ds(16, NCH - 16)],
                        didx_v.at[pl.ds(16, NCH - 16)])

    z = jnp.zeros((16,), jnp.float32)

    @pl.loop(0, 16)
    def _(r):
        for c in range(D // 16):
            buf_v[0, r, pl.ds(c * 16, 16)] = z

    @pl.loop(sid, N2 // 16, step=NS)
    def _(i):
        pltpu.sync_copy(buf_v.at[0, pl.ds(0, 16)], acc_sh.at[pl.ds(i * 16, 16)])

    plsc.subcore_barrier()

    def issue(j, b):
        pltpu.async_copy(y_hbm.at[sidx_v.at[j]], buf_v.at[b], sems[b])

    def wait_data(b):
        pltpu.make_async_copy(y_hbm.at[sidx_v.at[0]], buf_v.at[b],
                              sems[b]).wait()

    @pl.when(0 < ncw)
    def _():
        issue(0, 0)

    @pl.when(1 < ncw)
    def _():
        issue(1, 1)

    @pl.loop(0, (ncw + 1) // 2)
    def _(t):
        j0 = 2 * t
        for b in range(2):
            j = j0 + b

            @pl.when(j < ncw)
            def _():
                wait_data(b)
                pltpu.sync_copy(buf_v.at[b], acc_sh.at[didx_v.at[j]],
                                add=True)

                @pl.when(j + 2 < ncw)
                def _():
                    issue(j + 2, b)

    plsc.subcore_barrier()

    @pl.loop(sid, N2 // 16, step=NS)
    def _(i):
        pltpu.sync_copy(acc_sh.at[pl.ds(i * 16, 16)],
                        out_hbm.at[cid, pl.ds(i * 16, 16)])


def _mm_body(x_ref, w_ref, y_ref):
    y_ref[...] = jnp.dot(x_ref[...], w_ref[...],
                         preferred_element_type=jnp.float32)


def _scale_body(xw_ref, d0_ref, d1_ref, y_ref):
    dis = lax.rsqrt(d0_ref[...] + d1_ref[...] + 1.0)
    y_ref[...] = xw_ref[...] * dis


def _mid_body(p0_ref, p1_ref, y_ref, d0_ref, d1_ref, b_ref, w_ref, out_ref):
    dis = lax.rsqrt(d0_ref[...] + d1_ref[...] + 1.0)
    t = dis * (p0_ref[...] + p1_ref[...] + y_ref[...]) + b_ref[...]
    h = jnp.where(t > 0, t, jnp.exp(t) - 1.0)
    hw = jnp.dot(h, w_ref[...], preferred_element_type=jnp.float32)
    out_ref[...] = hw * dis


def _final_body(q0_ref, q1_ref, y_ref, d0_ref, d1_ref, b_ref, out_ref):
    dis = lax.rsqrt(d0_ref[...] + d1_ref[...] + 1.0)
    t = dis * (q0_ref[...] + q1_ref[...] + y_ref[...]) + b_ref[...]
    out_ref[...] = jnp.where(t > 0, t, jnp.exp(t) - 1.0)


def kernel(x, edge_index, W1, b1, W2, b2):
    ei = edge_index.astype(jnp.int32)
    src = ei[0]
    dst = ei[1]
    src1 = src.reshape(NC, NS, NCH, K)
    dst1 = dst.reshape(NC, NS, NCH, K)

    srcp = jnp.concatenate(
        [src.reshape(NW, EPW), jnp.full((NW, EPP - EPW), N, jnp.int32)],
        axis=1)
    dstp = jnp.concatenate(
        [dst.reshape(NW, EPW), jnp.full((NW, EPP - EPW), U, jnp.int32)],
        axis=1)
    src4 = srcp.reshape(NW, EPP // 128, 128)
    dst4 = dstp.reshape(NW, EPP // 128, 128)

    xp = jnp.concatenate([x, jnp.zeros((NP - N, D), x.dtype)], axis=0)
    R = 320
    grid = NP // R
    row_spec = pl.BlockSpec((R, D), lambda i: (i, 0))
    full_spec = pl.BlockSpec((D, D), lambda i: (0, 0))
    xw1 = pl.pallas_call(
        _mm_body,
        grid=(grid,),
        in_specs=[row_spec, full_spec],
        out_specs=row_spec,
        out_shape=jax.ShapeDtypeStruct((NP, D), jnp.float32),
    )(xp, W1)

    pos4, counts = pl.pallas_call(
        _pos_body,
        grid=(NW,),
        in_specs=[pl.BlockSpec((1, EPP // 128, 128), lambda i: (i, 0, 0))],
        out_specs=[pl.BlockSpec((1, EPP // 128, 128), lambda i: (i, 0, 0)),
                   pl.BlockSpec((1, 8, 128), lambda i: (i, 0, 0))],
        out_shape=[jax.ShapeDtypeStruct((NW, EPP // 128, 128), jnp.int32),
                   jax.ShapeDtypeStruct((NW, 8, 128), jnp.int32)],
    )(dst4)

    cnts = counts.reshape(NC, NS, 8, 128)
    deg_p, csrc, cdst = _prep_kernel(
        dst1, src4.reshape(NC, NS, EPP // 128, 128),
        dst4.reshape(NC, NS, EPP // 128, 128),
        pos4.reshape(NC, NS, EPP // 128, 128), cnts)

    degpad = jnp.zeros((NC, NP - N), jnp.float32)
    degp = jnp.concatenate([deg_p, degpad], axis=1)
    d0 = degp[0].reshape(NP, 1)
    d1 = degp[1].reshape(NP, 1)

    dcol_spec = pl.BlockSpec((R, 1), lambda i: (i, 0))
    bias_spec = pl.BlockSpec((1, D), lambda i: (0, 0))

    y1 = pl.pallas_call(
        _scale_body,
        grid=(grid,),
        in_specs=[row_spec, dcol_spec, dcol_spec],
        out_specs=row_spec,
        out_shape=jax.ShapeDtypeStruct((NP, D), jnp.float32),
    )(xw1, d0, d1)

    p = _edge_kernel(y1, src1, dst1)

    y2 = pl.pallas_call(
        _mid_body,
        grid=(grid,),
        in_specs=[row_spec, row_spec, row_spec, dcol_spec, dcol_spec,
                  bias_spec, full_spec],
        out_specs=row_spec,
        out_shape=jax.ShapeDtypeStruct((NP, D), jnp.float32),
    )(p[0], p[1], y1, d0, d1, b1.reshape(1, D), W2)

    csrc2 = csrc.reshape(NC, NS, SREG)[:, :, :EPW].reshape(NC, NS, NCH, K)
    cdst2 = cdst.reshape(NC, NS, SREG)[:, :, :EPW].reshape(NC, NS, NCH, K)
    q = _edge2_kernel(y2, csrc2, cdst2, cnts)

    RT = 128
    top_spec = pl.BlockSpec((RT, D), lambda i: (i, 0))
    top_dcol = pl.BlockSpec((RT, 1), lambda i: (i, 0))
    top_bias = pl.BlockSpec((1, D), lambda i: (0, 0))
    out_top = pl.pallas_call(
        _final_body,
        grid=(N2 // RT,),
        in_specs=[top_spec, top_spec, top_spec, top_dcol, top_dcol, top_bias],
        out_specs=top_spec,
        out_shape=jax.ShapeDtypeStruct((N2, D), jnp.float32),
    )(q[0], q[1], y2[:N2], d0[:N2], d1[:N2], b2.reshape(1, D))

    return jnp.concatenate([out_top[:U], x[U:]], axis=0)
